# baseline (speedup 1.0000x reference)
import functools

import jax
import jax.numpy as jnp
from jax.experimental import pallas as pl
from jax.experimental.pallas import tpu as pltpu
from jax.experimental.pallas import tpu_sc as plsc

BATCH, SEQ, DIM = 2, 4096, 1024
BASE_K, MAX_K = 256, 512

LANES = 128
N = SEQ
WIN = 256


def _stage_uni(win, sp, desc):
    n, L = win.shape
    y = win.reshape(-1, 2, sp, L)
    a, b = y[:, 0], y[:, 1]
    hi = jnp.maximum(a, b)
    lo = jnp.minimum(a, b)
    z = jnp.stack([hi, lo] if desc else [lo, hi], axis=1)
    return z.reshape(n, L)


def _stage_slice(x, sp, dpsize):
    n, L = x.shape
    y = x.reshape(-1, 2, dpsize // (2 * sp), 2, sp, L)
    a = y[:, :, :, 0]
    b = y[:, :, :, 1]
    hi = jnp.maximum(a, b)
    lo = jnp.minimum(a, b)
    top = jnp.stack([hi[:, 0], lo[:, 1]], axis=1)
    bot = jnp.stack([lo[:, 0], hi[:, 1]], axis=1)
    z = jnp.stack([top, bot], axis=3)
    return z.reshape(n, L)


def _stage_mask(x, sp, m_takemax):
    n, L = x.shape
    y = x.reshape(-1, 2, sp, L)
    partner = jnp.stack([y[:, 1], y[:, 0]], axis=1).reshape(n, L)
    hi = jnp.maximum(x, partner)
    lo = jnp.minimum(x, partner)
    return jnp.where(m_takemax, hi, lo)


def _stage_mask2(x, sp, ma):
    n, L = x.shape
    y = x.reshape(-1, 2, sp, L)
    a, b = y[:, 0], y[:, 1]
    hi = jnp.maximum(a, b)
    lo = jnp.minimum(a, b)
    top = jnp.where(ma, hi, lo)
    bot = jnp.where(ma, lo, hi)
    return jnp.stack([top, bot], axis=1).reshape(n, L)


def _stage_roll(x, sp, m_low, m_takemax):
    partner = jnp.where(m_low, jnp.roll(x, -sp, axis=0), jnp.roll(x, sp, axis=0))
    hi = jnp.maximum(x, partner)
    lo = jnp.minimum(x, partner)
    return jnp.where(m_takemax, hi, lo)


def _windowed(x, w, fn):
    parts = [fn(x[i * w:(i + 1) * w], i) for i in range(x.shape[0] // w)]
    return jnp.concatenate(parts, axis=0)


def _topk_body(scores_ref, out_ref):
    xn = scores_ref[...]
    x = jnp.swapaxes(xn, 0, 1)
    iota = jax.lax.broadcasted_iota(jnp.int32, (N, 1), 0)
    iwin = iota[:WIN]

    for p in range(1, 6):
        w = 8 << p

        def phase_fn(win, i, p=p):
            for t in range(p - 1, -1, -1):
                win = _stage_uni(win, 8 << t, i % 2 == 0)
            return win

        x = _windowed(x, w, phase_fn)

    for p in range(6, 10):
        for t in range(p - 1, 4, -1):
            sp = 8 << t
            if p <= 8:
                x = _stage_slice(x, sp, 8 << p)
            else:
                x = _stage_mask2(x, sp, (iota[:sp] & 1) == 0)
        if p <= 8:

            def tail_fn(win, i, p=p):
                desc = (((i * WIN) >> (3 + p)) & 1) == 0
                for t in range(4, -1, -1):
                    win = _stage_uni(win, 8 << t, desc)
                return win
        else:

            def tail_fn(win, i):
                for t in range(4, -1, -1):
                    sp = 8 << t
                    win = _stage_mask2(win, sp, (iwin[:sp] & 1) == 0)
                return win

        x = _windowed(x, WIN, tail_fn)

    for level in range(3):
        y = x.reshape(-1, 2, LANES)
        x = jnp.maximum(y[:, 0], y[:, 1])
        n = x.shape[0]
        k_il = n // 512
        it = iota[:n]
        big_ts = [t for t in range(8, -1, -1) if (k_il << t) > 128]
        for t in big_ts:
            sp = k_il << t
            if level < 2:
                x = _stage_mask2(x, sp, (iota[:sp] & 1) == 0)
            else:
                x = _stage_uni(x, sp, True)

        def resort_fn(win, i, k_il=k_il, level=level, nt=len(big_ts)):
            for t in range(8 - nt, -1, -1):
                sp = k_il << t
                if sp >= 8:
                    if level < 2:
                        win = _stage_mask2(win, sp, (iwin[:sp] & 1) == 0)
                    else:
                        win = _stage_uni(win, sp, True)
                else:
                    if level < 2:
                        m = ((iwin & sp) == 0) == ((iwin & 1) == 0)
                    else:
                        m = (iwin & sp) == 0
                    win = _stage_roll(win, sp, (iwin & sp) == 0, m)
            return win

        x = _windowed(x, min(WIN, n), resort_fn)

    out_ref[...] = jnp.swapaxes(x, 0, 1)


def _make_imp_sc():
    info = plsc.get_sparse_core_info()
    nc, ns = info.num_cores, info.num_subcores
    nw = nc * ns
    rows = BATCH * SEQ
    rpw = rows // nw
    mesh = plsc.VectorSubcoreMesh(core_axis_name="c", subcore_axis_name="s")

    @functools.partial(
        pl.kernel, mesh=mesh,
        out_type=jax.ShapeDtypeStruct((rows,), jnp.int32),
        scratch_types=[
            pltpu.VMEM((16, DIM), jnp.float32),
            pltpu.VMEM((DIM,), jnp.float32),
            pltpu.VMEM((16,), jnp.float32),
            pltpu.VMEM((16,), jnp.int32),
        ],
    )
    def imp_sc(q_hbm, w_hbm, b_hbm, out_hbm, qv, wv, bv, outv):
        wid = jax.lax.axis_index("s") * nc + jax.lax.axis_index("c")
        base = wid * rpw
        pltpu.sync_copy(w_hbm, wv)
        pltpu.sync_copy(b_hbm, bv)
        lanes = jax.lax.iota(jnp.int32, 16)

        def chunk_body(ch, carry):
            row0 = base + ch * 16
            pltpu.sync_copy(q_hbm.at[pl.ds(row0, 16)], qv)
            ks = jnp.zeros((16,), jnp.int32)
            for r in range(16):
                acc = qv[r, pl.ds(0, 16)] * wv[pl.ds(0, 16)]
                for jg in range(1, DIM // 16):
                    acc = acc + qv[r, pl.ds(jg * 16, 16)] * wv[pl.ds(jg * 16, 16)]
                for s in (8, 4, 2, 1):
                    acc = acc + acc.at[lanes ^ s].get(mode="promise_in_bounds")
                zv = acc + bv[...]
                sig = 1.0 / (1.0 + jnp.exp(-zv))
                kv = (BASE_K + (MAX_K - BASE_K) * sig).astype(jnp.int32)
                ks = jnp.where(lanes == r, kv, ks)
            outv[...] = ks
            pltpu.sync_copy(outv, out_hbm.at[pl.ds(row0, 16)])
            return carry

        jax.lax.fori_loop(0, rpw // 16, chunk_body, 0)

    return imp_sc


_imp_sc = _make_imp_sc()


def kernel(Q, scores, W, b):
    rows = BATCH * SEQ
    scores_f = scores.reshape(rows, SEQ)
    q_f = Q.reshape(rows, DIM)

    vals = pl.pallas_call(
        _topk_body,
        grid=(rows // LANES,),
        in_specs=[pl.BlockSpec((LANES, SEQ), lambda i: (i, 0))],
        out_specs=pl.BlockSpec((LANES, MAX_K), lambda i: (i, 0)),
        out_shape=jax.ShapeDtypeStruct((rows, MAX_K), jnp.float32),
    )(scores_f)

    w_vec = W.reshape(DIM)
    b16 = jnp.broadcast_to(b, (16,)).astype(jnp.float32)
    k_out = _imp_sc(q_f, w_vec, b16)

    return (vals.reshape(BATCH, SEQ, MAX_K), k_out.reshape(BATCH, SEQ))


# speedup vs baseline: 1.0820x; 1.0820x over previous
import functools

import jax
import jax.numpy as jnp
from jax.experimental import pallas as pl
from jax.experimental.pallas import tpu as pltpu
from jax.experimental.pallas import tpu_sc as plsc

BATCH, SEQ, DIM = 2, 4096, 1024
BASE_K, MAX_K = 256, 512

LANES = 128
N = SEQ
WIN = 256


def _stage_uni(win, sp, desc):
    n, L = win.shape
    y = win.reshape(-1, 2, sp, L)
    a, b = y[:, 0], y[:, 1]
    hi = jnp.maximum(a, b)
    lo = jnp.minimum(a, b)
    z = jnp.stack([hi, lo] if desc else [lo, hi], axis=1)
    return z.reshape(n, L)


def _stage_slice(x, sp, dpsize):
    n, L = x.shape
    y = x.reshape(-1, 2, dpsize // (2 * sp), 2, sp, L)
    a = y[:, :, :, 0]
    b = y[:, :, :, 1]
    hi = jnp.maximum(a, b)
    lo = jnp.minimum(a, b)
    top = jnp.stack([hi[:, 0], lo[:, 1]], axis=1)
    bot = jnp.stack([lo[:, 0], hi[:, 1]], axis=1)
    z = jnp.stack([top, bot], axis=3)
    return z.reshape(n, L)


def _stage_mask(x, sp, m_takemax):
    n, L = x.shape
    y = x.reshape(-1, 2, sp, L)
    partner = jnp.stack([y[:, 1], y[:, 0]], axis=1).reshape(n, L)
    hi = jnp.maximum(x, partner)
    lo = jnp.minimum(x, partner)
    return jnp.where(m_takemax, hi, lo)


def _stage_roll(x, sp, m_low, m_takemax):
    partner = jnp.where(m_low, jnp.roll(x, -sp, axis=0), jnp.roll(x, sp, axis=0))
    hi = jnp.maximum(x, partner)
    lo = jnp.minimum(x, partner)
    return jnp.where(m_takemax, hi, lo)


def _windowed(x, w, fn):
    parts = [fn(x[i * w:(i + 1) * w], i) for i in range(x.shape[0] // w)]
    return jnp.concatenate(parts, axis=0)


def _topk_body(scores_ref, out_ref):
    xn = scores_ref[...]
    x = jnp.swapaxes(xn, 0, 1)
    iota = jax.lax.broadcasted_iota(jnp.int32, (N, 1), 0)
    iwin = iota[:WIN]

    for p in range(1, 6):
        w = 8 << p

        def phase_fn(win, i, p=p):
            for t in range(p - 1, -1, -1):
                win = _stage_uni(win, 8 << t, i % 2 == 0)
            return win

        x = _windowed(x, w, phase_fn)

    for p in range(6, 10):
        for t in range(p - 1, 4, -1):
            sp = 8 << t
            if p <= 8:
                x = _stage_slice(x, sp, 8 << p)
            else:
                m = ((iota & sp) == 0) == ((iota & 1) == 0)
                x = _stage_mask(x, sp, m)
        if p <= 8:

            def tail_fn(win, i, p=p):
                desc = (((i * WIN) >> (3 + p)) & 1) == 0
                for t in range(4, -1, -1):
                    win = _stage_uni(win, 8 << t, desc)
                return win
        else:

            def tail_fn(win, i):
                for t in range(4, -1, -1):
                    sp = 8 << t
                    m = ((iwin & sp) == 0) == ((iwin & 1) == 0)
                    win = _stage_mask(win, sp, m)
                return win

        x = _windowed(x, WIN, tail_fn)

    for level in range(3):
        x = jnp.maximum(x, jnp.roll(x, -1, axis=0)).reshape(-1, 2, LANES)[:, 0]
        n = x.shape[0]
        k_il = n // 512
        it = iota[:n]
        big_ts = [t for t in range(8, -1, -1) if (k_il << t) > 128]
        for t in big_ts:
            sp = k_il << t
            if level < 2:
                m = ((it & sp) == 0) == ((it & 1) == 0)
            else:
                m = (it & sp) == 0
            x = _stage_mask(x, sp, m)

        def resort_fn(win, i, k_il=k_il, level=level, nt=len(big_ts)):
            for t in range(8 - nt, -1, -1):
                sp = k_il << t
                if level < 2:
                    m = ((iwin & sp) == 0) == ((iwin & 1) == 0)
                else:
                    m = (iwin & sp) == 0
                if sp >= 8:
                    win = _stage_mask(win, sp, m)
                else:
                    win = _stage_roll(win, sp, (iwin & sp) == 0, m)
            return win

        x = _windowed(x, min(WIN, n), resort_fn)

    out_ref[...] = jnp.swapaxes(x, 0, 1)


def _make_imp_sc():
    info = plsc.get_sparse_core_info()
    nc, ns = info.num_cores, info.num_subcores
    nw = nc * ns
    rows = BATCH * SEQ
    rpw = rows // nw
    mesh = plsc.VectorSubcoreMesh(core_axis_name="c", subcore_axis_name="s")

    @functools.partial(
        pl.kernel, mesh=mesh,
        out_type=jax.ShapeDtypeStruct((rows,), jnp.int32),
        scratch_types=[
            pltpu.VMEM((16, DIM), jnp.float32),
            pltpu.VMEM((DIM,), jnp.float32),
            pltpu.VMEM((16,), jnp.float32),
            pltpu.VMEM((16,), jnp.int32),
        ],
    )
    def imp_sc(q_hbm, w_hbm, b_hbm, out_hbm, qv, wv, bv, outv):
        wid = jax.lax.axis_index("s") * nc + jax.lax.axis_index("c")
        base = wid * rpw
        pltpu.sync_copy(w_hbm, wv)
        pltpu.sync_copy(b_hbm, bv)
        lanes = jax.lax.iota(jnp.int32, 16)

        def chunk_body(ch, carry):
            row0 = base + ch * 16
            pltpu.sync_copy(q_hbm.at[pl.ds(row0, 16)], qv)
            ks = jnp.zeros((16,), jnp.int32)
            for r in range(16):
                acc = qv[r, pl.ds(0, 16)] * wv[pl.ds(0, 16)]
                for jg in range(1, DIM // 16):
                    acc = acc + qv[r, pl.ds(jg * 16, 16)] * wv[pl.ds(jg * 16, 16)]
                for s in (8, 4, 2, 1):
                    acc = acc + acc.at[lanes ^ s].get(mode="promise_in_bounds")
                zv = acc + bv[...]
                sig = 1.0 / (1.0 + jnp.exp(-zv))
                kv = (BASE_K + (MAX_K - BASE_K) * sig).astype(jnp.int32)
                ks = jnp.where(lanes == r, kv, ks)
            outv[...] = ks
            pltpu.sync_copy(outv, out_hbm.at[pl.ds(row0, 16)])
            return carry

        jax.lax.fori_loop(0, rpw // 16, chunk_body, 0)

    return imp_sc


_imp_sc = _make_imp_sc()


def kernel(Q, scores, W, b):
    rows = BATCH * SEQ
    scores_f = scores.reshape(rows, SEQ)
    q_f = Q.reshape(rows, DIM)

    vals = pl.pallas_call(
        _topk_body,
        grid=(rows // LANES,),
        in_specs=[pl.BlockSpec((LANES, SEQ), lambda i: (i, 0))],
        out_specs=pl.BlockSpec((LANES, MAX_K), lambda i: (i, 0)),
        out_shape=jax.ShapeDtypeStruct((rows, MAX_K), jnp.float32),
    )(scores_f)

    w_vec = W.reshape(DIM)
    b16 = jnp.broadcast_to(b, (16,)).astype(jnp.float32)
    k_out = _imp_sc(q_f, w_vec, b16)

    return (vals.reshape(BATCH, SEQ, MAX_K), k_out.reshape(BATCH, SEQ))
